# baseline (speedup 1.0000x reference)
import functools

import jax
import jax.numpy as jnp
from jax import lax
from jax.experimental import pallas as pl
from jax.experimental.pallas import tpu as pltpu
from jax.experimental.pallas import tpu_sc as plsc

D_MODEL = 1024
MAX_LEN = 300
SEQ = 8192

NUM_CORES = 2
NUM_SUBCORES = 16
NUM_WORKERS = NUM_CORES * NUM_SUBCORES
B_PER_W = SEQ // NUM_WORKERS
CHUNK = 32
N_CHUNKS = B_PER_W // CHUNK
NBUF = 2
LANES = 16


def _sc_body(src_hbm, sil_hbm, pe_hbm, out_hbm, sil_v, idx_v, srcbuf_v, pebuf_v,
             sem_src, sem_pe, sem_out):
    wid = lax.axis_index("s") * NUM_CORES + lax.axis_index("c")
    base = wid * B_PER_W

    def start_src(c, b):
        off = base + c * CHUNK
        pltpu.async_copy(src_hbm.at[pl.ds(off, CHUNK)], srcbuf_v.at[b],
                         sem_src.at[b])

    def start_pe(c, b):
        pltpu.async_copy(pe_hbm.at[idx_v.at[pl.ds(c * CHUNK, CHUNK)]],
                         pebuf_v.at[b], sem_pe.at[b])

    def wait_loads(c, b):
        off = base + c * CHUNK
        pltpu.make_async_copy(src_hbm.at[pl.ds(off, CHUNK)], srcbuf_v.at[b],
                              sem_src.at[b]).wait()
        pltpu.make_async_copy(pe_hbm.at[idx_v.at[pl.ds(c * CHUNK, CHUNK)]],
                              pebuf_v.at[b], sem_pe.at[b]).wait()

    def do_add(b):
        @plsc.parallel_loop(0, CHUNK, unroll=2)
        def _add_row(r):
            for k in range(D_MODEL // (2 * LANES)):
                pe_words = pebuf_v[b, r, pl.ds(k * LANES, LANES)]
                lo = lax.bitcast_convert_type(pe_words << 16, jnp.float32)
                hi = lax.bitcast_convert_type(
                    pe_words & jnp.int32(-65536), jnp.float32
                )
                sl_lo = pl.ds(k * 2 * LANES, LANES)
                sl_hi = pl.ds(k * 2 * LANES + LANES, LANES)
                plsc.addupdate(srcbuf_v.at[b, r, sl_lo], lo)
                plsc.addupdate(srcbuf_v.at[b, r, sl_hi], hi)

    start_src(0, 0)

    pltpu.sync_copy(sil_hbm.at[pl.ds(base, B_PER_W)], sil_v)

    for k in range(B_PER_W // LANES):
        s = sil_v[pl.ds(k * LANES, LANES)]
        idx_v[pl.ds(k * LANES, LANES)] = jnp.where(
            s > 0, jnp.minimum(s, MAX_LEN - 1), MAX_LEN
        )

    start_pe(0, 0)
    for c in range(1, NBUF):
        start_src(c, c)
        start_pe(c, c)

    def wait_store(c, b):
        pltpu.make_async_copy(srcbuf_v.at[b],
                              out_hbm.at[pl.ds(base + c * CHUNK, CHUNK)],
                              sem_out.at[b]).wait()

    for c in range(N_CHUNKS):
        cur = c % NBUF
        if c >= 1 and c - 1 + NBUF < N_CHUNKS:
            b = (c - 1) % NBUF
            wait_store(c - 1, b)
            start_src(c - 1 + NBUF, b)
            start_pe(c - 1 + NBUF, b)
        wait_loads(c, cur)
        do_add(cur)
        pltpu.async_copy(srcbuf_v.at[cur],
                         out_hbm.at[pl.ds(base + c * CHUNK, CHUNK)],
                         sem_out.at[cur])
    for c in range(max(0, N_CHUNKS - NBUF), N_CHUNKS):
        wait_store(c, c % NBUF)


@jax.jit
def _run(src2d, sil, pe_bf16):
    mesh = plsc.VectorSubcoreMesh(core_axis_name="c", subcore_axis_name="s")
    fn = pl.kernel(
        _sc_body,
        out_type=jax.ShapeDtypeStruct((SEQ, D_MODEL), jnp.float32),
        mesh=mesh,
        scratch_types=[
            pltpu.VMEM((B_PER_W,), jnp.int32),
            pltpu.VMEM((B_PER_W,), jnp.int32),
            pltpu.VMEM((NBUF, CHUNK, D_MODEL), jnp.float32),
            pltpu.VMEM((NBUF, CHUNK, D_MODEL // 2), jnp.int32),
            pltpu.SemaphoreType.DMA((NBUF,)),
            pltpu.SemaphoreType.DMA((NBUF,)),
            pltpu.SemaphoreType.DMA((NBUF,)),
        ],
    )
    return fn(src2d, sil, pe_bf16)


def kernel(src, silence, pe):
    src2d = src.reshape(SEQ, D_MODEL)
    sil = silence.astype(jnp.int32)
    pe_pad = jnp.concatenate(
        [pe.astype(jnp.float32), jnp.zeros((1, D_MODEL), jnp.float32)], axis=0
    )
    pe_bf16 = (
        pe_pad.reshape(MAX_LEN + 1, D_MODEL // 32, 2, LANES)
        .swapaxes(2, 3)
        .reshape(MAX_LEN + 1, D_MODEL // 2, 2)
        .astype(jnp.bfloat16)
    )
    pe_i32 = jax.lax.bitcast_convert_type(pe_bf16, jnp.int32)
    out = _run(src2d, sil, pe_i32)
    return out.reshape(1, SEQ, D_MODEL)


# speedup vs baseline: 1.0171x; 1.0171x over previous
import functools

import jax
import jax.numpy as jnp
from jax import lax
from jax.experimental import pallas as pl
from jax.experimental.pallas import tpu as pltpu
from jax.experimental.pallas import tpu_sc as plsc

D_MODEL = 1024
MAX_LEN = 300
SEQ = 8192

NUM_CORES = 2
NUM_SUBCORES = 16
NUM_WORKERS = NUM_CORES * NUM_SUBCORES
B_PER_W = SEQ // NUM_WORKERS
CHUNK = 32
N_CHUNKS = B_PER_W // CHUNK
NBUF = 2
PEBUF = 3
LANES = 16


def _sc_body(src_hbm, sil_hbm, pe_hbm, out_hbm, sil_v, idx_v, srcbuf_v, pebuf_v,
             sem_src, sem_pe, sem_out):
    wid = lax.axis_index("s") * NUM_CORES + lax.axis_index("c")
    base = wid * B_PER_W

    def start_src(c, b):
        off = base + c * CHUNK
        pltpu.async_copy(src_hbm.at[pl.ds(off, CHUNK)], srcbuf_v.at[b],
                         sem_src.at[b])

    def start_pe(c):
        b = c % PEBUF
        pltpu.async_copy(pe_hbm.at[idx_v.at[pl.ds(c * CHUNK, CHUNK)]],
                         pebuf_v.at[b], sem_pe.at[b])

    def wait_loads(c, b):
        off = base + c * CHUNK
        pltpu.make_async_copy(src_hbm.at[pl.ds(off, CHUNK)], srcbuf_v.at[b],
                              sem_src.at[b]).wait()
        pltpu.make_async_copy(
            pe_hbm.at[idx_v.at[pl.ds(c * CHUNK, CHUNK)]],
            pebuf_v.at[c % PEBUF], sem_pe.at[c % PEBUF]
        ).wait()

    def do_add(c, b):
        pb = c % PEBUF

        @plsc.parallel_loop(0, CHUNK, unroll=2)
        def _add_row(r):
            for k in range(D_MODEL // (2 * LANES)):
                pe_words = pebuf_v[pb, r, pl.ds(k * LANES, LANES)]
                lo = lax.bitcast_convert_type(pe_words << 16, jnp.float32)
                hi = lax.bitcast_convert_type(
                    pe_words & jnp.int32(-65536), jnp.float32
                )
                sl_lo = pl.ds(k * 2 * LANES, LANES)
                sl_hi = pl.ds(k * 2 * LANES + LANES, LANES)
                plsc.addupdate(srcbuf_v.at[b, r, sl_lo], lo)
                plsc.addupdate(srcbuf_v.at[b, r, sl_hi], hi)

    start_src(0, 0)

    pltpu.sync_copy(sil_hbm.at[pl.ds(base, B_PER_W)], sil_v)

    for k in range(B_PER_W // LANES):
        s = sil_v[pl.ds(k * LANES, LANES)]
        idx_v[pl.ds(k * LANES, LANES)] = jnp.where(
            s > 0, jnp.minimum(s, MAX_LEN - 1), MAX_LEN
        )

    start_pe(0)
    for c in range(1, NBUF):
        start_src(c, c)
    for c in range(1, PEBUF):
        start_pe(c)

    def wait_store(c, b):
        pltpu.make_async_copy(srcbuf_v.at[b],
                              out_hbm.at[pl.ds(base + c * CHUNK, CHUNK)],
                              sem_out.at[b]).wait()

    for c in range(N_CHUNKS):
        cur = c % NBUF
        if c >= 1 and c - 1 + NBUF < N_CHUNKS:
            b = (c - 1) % NBUF
            wait_store(c - 1, b)
            start_src(c - 1 + NBUF, b)
        if c >= 1 and c - 1 + PEBUF < N_CHUNKS:
            start_pe(c - 1 + PEBUF)
        wait_loads(c, cur)
        do_add(c, cur)
        pltpu.async_copy(srcbuf_v.at[cur],
                         out_hbm.at[pl.ds(base + c * CHUNK, CHUNK)],
                         sem_out.at[cur])
    for c in range(max(0, N_CHUNKS - NBUF), N_CHUNKS):
        wait_store(c, c % NBUF)


@jax.jit
def _run(src2d, sil, pe_bf16):
    mesh = plsc.VectorSubcoreMesh(core_axis_name="c", subcore_axis_name="s")
    fn = pl.kernel(
        _sc_body,
        out_type=jax.ShapeDtypeStruct((SEQ, D_MODEL), jnp.float32),
        mesh=mesh,
        scratch_types=[
            pltpu.VMEM((B_PER_W,), jnp.int32),
            pltpu.VMEM((B_PER_W,), jnp.int32),
            pltpu.VMEM((NBUF, CHUNK, D_MODEL), jnp.float32),
            pltpu.VMEM((PEBUF, CHUNK, D_MODEL // 2), jnp.int32),
            pltpu.SemaphoreType.DMA((NBUF,)),
            pltpu.SemaphoreType.DMA((PEBUF,)),
            pltpu.SemaphoreType.DMA((NBUF,)),
        ],
    )
    return fn(src2d, sil, pe_bf16)


def kernel(src, silence, pe):
    src2d = src.reshape(SEQ, D_MODEL)
    sil = silence.astype(jnp.int32)
    pe_bf16 = (
        pe.astype(jnp.float32)
        .reshape(MAX_LEN, D_MODEL // 32, 2, LANES)
        .swapaxes(2, 3)
        .reshape(MAX_LEN, D_MODEL // 2, 2)
        .astype(jnp.bfloat16)
    )
    pe_i32 = jnp.concatenate(
        [
            jax.lax.bitcast_convert_type(pe_bf16, jnp.int32),
            jnp.zeros((1, D_MODEL // 2), jnp.int32),
        ],
        axis=0,
    )
    out = _run(src2d, sil, pe_i32)
    return out.reshape(1, SEQ, D_MODEL)


# speedup vs baseline: 1.1432x; 1.1239x over previous
import functools

import jax
import jax.numpy as jnp
from jax import lax
from jax.experimental import pallas as pl
from jax.experimental.pallas import tpu as pltpu
from jax.experimental.pallas import tpu_sc as plsc

D_MODEL = 1024
MAX_LEN = 300
SEQ = 8192

NUM_CORES = 2
NUM_SUBCORES = 16
NUM_WORKERS = NUM_CORES * NUM_SUBCORES
B_PER_W = SEQ // NUM_WORKERS
CHUNK = 32
N_CHUNKS = B_PER_W // CHUNK
NBUF = 2
PEBUF = 3
LANES = 16


def _sc_body(src_hbm, sil_hbm, pe_hbm, out_hbm, sil_v, idx_v, srcbuf_v, pebuf_v,
             sem_src, sem_pe, sem_out):
    wid = lax.axis_index("s") * NUM_CORES + lax.axis_index("c")
    base = wid * B_PER_W

    def start_src(c, b):
        off = base + c * CHUNK
        pltpu.async_copy(src_hbm.at[pl.ds(off, CHUNK)], srcbuf_v.at[b],
                         sem_src.at[b])

    def start_pe(c, b):
        pltpu.async_copy(pe_hbm.at[idx_v.at[pl.ds(c * CHUNK, CHUNK)]],
                         pebuf_v.at[b], sem_pe.at[b])

    def wait_loads(c, b, pb):
        off = base + c * CHUNK
        pltpu.make_async_copy(src_hbm.at[pl.ds(off, CHUNK)], srcbuf_v.at[b],
                              sem_src.at[b]).wait()
        pltpu.make_async_copy(
            pe_hbm.at[idx_v.at[pl.ds(c * CHUNK, CHUNK)]],
            pebuf_v.at[pb], sem_pe.at[pb]
        ).wait()

    def do_add(c, b):
        pb = lax.rem(c, PEBUF)

        @plsc.parallel_loop(0, CHUNK, unroll=4)
        def _add_row(r):
            for k in range(D_MODEL // (2 * LANES)):
                pe_words = pebuf_v[pb, r, pl.ds(k * LANES, LANES)]
                lo = lax.bitcast_convert_type(pe_words << 16, jnp.float32)
                hi = lax.bitcast_convert_type(
                    pe_words & jnp.int32(-65536), jnp.float32
                )
                sl_lo = pl.ds(k * 2 * LANES, LANES)
                sl_hi = pl.ds(k * 2 * LANES + LANES, LANES)
                plsc.addupdate(srcbuf_v.at[b, r, sl_lo], lo)
                plsc.addupdate(srcbuf_v.at[b, r, sl_hi], hi)

    start_src(0, 0)

    pltpu.sync_copy(sil_hbm.at[pl.ds(base, B_PER_W)], sil_v)

    for k in range(B_PER_W // LANES):
        s = sil_v[pl.ds(k * LANES, LANES)]
        idx_v[pl.ds(k * LANES, LANES)] = jnp.where(
            s > 0, jnp.minimum(s, MAX_LEN - 1), MAX_LEN
        )

    start_pe(0, 0)
    for c in range(1, NBUF):
        start_src(c, c)
    for c in range(1, PEBUF):
        start_pe(c, c)

    def wait_store(c, b):
        pltpu.make_async_copy(srcbuf_v.at[b],
                              out_hbm.at[pl.ds(base + c * CHUNK, CHUNK)],
                              sem_out.at[b]).wait()

    def chunk_body(c, _):
        cur = jnp.bitwise_and(c, NBUF - 1)
        prv = jnp.bitwise_xor(cur, 1)

        @pl.when(jnp.logical_and(c >= 1, c + 1 < N_CHUNKS))
        def _():
            wait_store(c - 1, prv)
            start_src(c - 1 + NBUF, prv)

        @pl.when(jnp.logical_and(c >= 1, c - 1 + PEBUF < N_CHUNKS))
        def _():
            start_pe(c - 1 + PEBUF, lax.rem(c - 1 + PEBUF, PEBUF))

        wait_loads(c, cur, lax.rem(c, PEBUF))
        do_add(c, cur)
        pltpu.async_copy(srcbuf_v.at[cur],
                         out_hbm.at[pl.ds(base + c * CHUNK, CHUNK)],
                         sem_out.at[cur])
        return 0

    lax.fori_loop(0, N_CHUNKS, chunk_body, 0)
    for c in range(max(0, N_CHUNKS - NBUF), N_CHUNKS):
        wait_store(c, c % NBUF)


@jax.jit
def _run(src2d, sil, pe_bf16):
    mesh = plsc.VectorSubcoreMesh(core_axis_name="c", subcore_axis_name="s")
    fn = pl.kernel(
        _sc_body,
        out_type=jax.ShapeDtypeStruct((SEQ, D_MODEL), jnp.float32),
        mesh=mesh,
        scratch_types=[
            pltpu.VMEM((B_PER_W,), jnp.int32),
            pltpu.VMEM((B_PER_W,), jnp.int32),
            pltpu.VMEM((NBUF, CHUNK, D_MODEL), jnp.float32),
            pltpu.VMEM((PEBUF, CHUNK, D_MODEL // 2), jnp.int32),
            pltpu.SemaphoreType.DMA((NBUF,)),
            pltpu.SemaphoreType.DMA((PEBUF,)),
            pltpu.SemaphoreType.DMA((NBUF,)),
        ],
    )
    return fn(src2d, sil, pe_bf16)


def kernel(src, silence, pe):
    src2d = src.reshape(SEQ, D_MODEL)
    sil = silence.astype(jnp.int32)
    pe_bf16 = (
        pe.astype(jnp.float32)
        .reshape(MAX_LEN, D_MODEL // 32, 2, LANES)
        .swapaxes(2, 3)
        .reshape(MAX_LEN, D_MODEL // 2, 2)
        .astype(jnp.bfloat16)
    )
    pe_i32 = jnp.concatenate(
        [
            jax.lax.bitcast_convert_type(pe_bf16, jnp.int32),
            jnp.zeros((1, D_MODEL // 2), jnp.int32),
        ],
        axis=0,
    )
    out = _run(src2d, sil, pe_i32)
    return out.reshape(1, SEQ, D_MODEL)


# speedup vs baseline: 1.1599x; 1.0147x over previous
import functools

import jax
import jax.numpy as jnp
from jax import lax
from jax.experimental import pallas as pl
from jax.experimental.pallas import tpu as pltpu
from jax.experimental.pallas import tpu_sc as plsc

D_MODEL = 1024
MAX_LEN = 300
SEQ = 8192

NUM_CORES = 2
NUM_SUBCORES = 16
NUM_WORKERS = NUM_CORES * NUM_SUBCORES
B_PER_W = SEQ // NUM_WORKERS
CHUNK = 32
N_CHUNKS = B_PER_W // CHUNK
NBUF = 2
PEBUF = 3
LANES = 16


def _sc_body(src_hbm, sil_hbm, pe_hbm, out_hbm, sil_v, idx_v, srcbuf_v, pebuf_v,
             sem_src, sem_pe, sem_out):
    wid = lax.axis_index("s") * NUM_CORES + lax.axis_index("c")
    base = wid * B_PER_W

    def start_src(c, b):
        off = base + c * CHUNK
        pltpu.async_copy(src_hbm.at[pl.ds(off, CHUNK)], srcbuf_v.at[b],
                         sem_src.at[b])

    def start_pe(c, b):
        pltpu.async_copy(pe_hbm.at[idx_v.at[pl.ds(c * CHUNK, CHUNK)]],
                         pebuf_v.at[b], sem_pe.at[b])

    def wait_loads(c, b, pb):
        off = base + c * CHUNK
        pltpu.make_async_copy(src_hbm.at[pl.ds(off, CHUNK)], srcbuf_v.at[b],
                              sem_src.at[b]).wait()
        pltpu.make_async_copy(
            pe_hbm.at[idx_v.at[pl.ds(c * CHUNK, CHUNK)]],
            pebuf_v.at[pb], sem_pe.at[pb]
        ).wait()

    def do_add(c, b):
        pb = lax.rem(c, PEBUF)

        @plsc.parallel_loop(0, CHUNK, unroll=8)
        def _add_row(r):
            for k in range(D_MODEL // (2 * LANES)):
                pe_words = pebuf_v[pb, r, pl.ds(k * LANES, LANES)]
                lo = lax.bitcast_convert_type(pe_words << 16, jnp.float32)
                hi = lax.bitcast_convert_type(
                    pe_words & jnp.int32(-65536), jnp.float32
                )
                sl_lo = pl.ds(k * 2 * LANES, LANES)
                sl_hi = pl.ds(k * 2 * LANES + LANES, LANES)
                plsc.addupdate(srcbuf_v.at[b, r, sl_lo], lo)
                plsc.addupdate(srcbuf_v.at[b, r, sl_hi], hi)

    start_src(0, 0)

    pltpu.sync_copy(sil_hbm.at[pl.ds(base, B_PER_W)], sil_v)

    for k in range(B_PER_W // LANES):
        s = sil_v[pl.ds(k * LANES, LANES)]
        idx_v[pl.ds(k * LANES, LANES)] = jnp.where(
            s > 0, jnp.minimum(s, MAX_LEN - 1), MAX_LEN
        )

    start_pe(0, 0)
    for c in range(1, NBUF):
        start_src(c, c)
    for c in range(1, PEBUF):
        start_pe(c, c)

    def wait_store(c, b):
        pltpu.make_async_copy(srcbuf_v.at[b],
                              out_hbm.at[pl.ds(base + c * CHUNK, CHUNK)],
                              sem_out.at[b]).wait()

    def chunk_body(c, _):
        cur = jnp.bitwise_and(c, NBUF - 1)
        prv = jnp.bitwise_xor(cur, 1)

        @pl.when(jnp.logical_and(c >= 1, c + 1 < N_CHUNKS))
        def _():
            wait_store(c - 1, prv)
            start_src(c - 1 + NBUF, prv)

        @pl.when(jnp.logical_and(c >= 1, c - 1 + PEBUF < N_CHUNKS))
        def _():
            start_pe(c - 1 + PEBUF, lax.rem(c - 1 + PEBUF, PEBUF))

        wait_loads(c, cur, lax.rem(c, PEBUF))
        do_add(c, cur)
        pltpu.async_copy(srcbuf_v.at[cur],
                         out_hbm.at[pl.ds(base + c * CHUNK, CHUNK)],
                         sem_out.at[cur])
        return 0

    lax.fori_loop(0, N_CHUNKS, chunk_body, 0)
    for c in range(max(0, N_CHUNKS - NBUF), N_CHUNKS):
        wait_store(c, c % NBUF)


@jax.jit
def _run(src2d, sil, pe_bf16):
    mesh = plsc.VectorSubcoreMesh(core_axis_name="c", subcore_axis_name="s")
    fn = pl.kernel(
        _sc_body,
        out_type=jax.ShapeDtypeStruct((SEQ, D_MODEL), jnp.float32),
        mesh=mesh,
        scratch_types=[
            pltpu.VMEM((B_PER_W,), jnp.int32),
            pltpu.VMEM((B_PER_W,), jnp.int32),
            pltpu.VMEM((NBUF, CHUNK, D_MODEL), jnp.float32),
            pltpu.VMEM((PEBUF, CHUNK, D_MODEL // 2), jnp.int32),
            pltpu.SemaphoreType.DMA((NBUF,)),
            pltpu.SemaphoreType.DMA((PEBUF,)),
            pltpu.SemaphoreType.DMA((NBUF,)),
        ],
    )
    return fn(src2d, sil, pe_bf16)


def kernel(src, silence, pe):
    src2d = src.reshape(SEQ, D_MODEL)
    sil = silence.astype(jnp.int32)
    pe_bf16 = (
        pe.astype(jnp.float32)
        .reshape(MAX_LEN, D_MODEL // 32, 2, LANES)
        .swapaxes(2, 3)
        .reshape(MAX_LEN, D_MODEL // 2, 2)
        .astype(jnp.bfloat16)
    )
    pe_i32 = jnp.concatenate(
        [
            jax.lax.bitcast_convert_type(pe_bf16, jnp.int32),
            jnp.zeros((1, D_MODEL // 2), jnp.int32),
        ],
        axis=0,
    )
    out = _run(src2d, sil, pe_i32)
    return out.reshape(1, SEQ, D_MODEL)


# speedup vs baseline: 1.1834x; 1.0203x over previous
import functools

import jax
import jax.numpy as jnp
from jax import lax
from jax.experimental import pallas as pl
from jax.experimental.pallas import tpu as pltpu
from jax.experimental.pallas import tpu_sc as plsc

D_MODEL = 1024
MAX_LEN = 300
SEQ = 8192

NUM_CORES = 2
NUM_SUBCORES = 16
NUM_WORKERS = NUM_CORES * NUM_SUBCORES
B_PER_W = SEQ // NUM_WORKERS
CHUNK = 16
N_CHUNKS = B_PER_W // CHUNK
NBUF = 4
PEBUF = 5
LANES = 16


def _sc_body(src_hbm, sil_hbm, pe_hbm, out_hbm, sil_v, idx_v, srcbuf_v, pebuf_v,
             sem_src, sem_pe, sem_out):
    wid = lax.axis_index("s") * NUM_CORES + lax.axis_index("c")
    base = wid * B_PER_W

    def start_src(c, b):
        off = base + c * CHUNK
        pltpu.async_copy(src_hbm.at[pl.ds(off, CHUNK)], srcbuf_v.at[b],
                         sem_src.at[b])

    def start_pe(c, b):
        pltpu.async_copy(pe_hbm.at[idx_v.at[pl.ds(c * CHUNK, CHUNK)]],
                         pebuf_v.at[b], sem_pe.at[b])

    def wait_loads(c, b, pb):
        off = base + c * CHUNK
        pltpu.make_async_copy(src_hbm.at[pl.ds(off, CHUNK)], srcbuf_v.at[b],
                              sem_src.at[b]).wait()
        pltpu.make_async_copy(
            pe_hbm.at[idx_v.at[pl.ds(c * CHUNK, CHUNK)]],
            pebuf_v.at[pb], sem_pe.at[pb]
        ).wait()

    def do_add(c, b):
        pb = lax.rem(c, PEBUF)

        @plsc.parallel_loop(0, CHUNK, unroll=8)
        def _add_row(r):
            for k in range(D_MODEL // (2 * LANES)):
                pe_words = pebuf_v[pb, r, pl.ds(k * LANES, LANES)]
                lo = lax.bitcast_convert_type(pe_words << 16, jnp.float32)
                hi = lax.bitcast_convert_type(
                    pe_words & jnp.int32(-65536), jnp.float32
                )
                sl_lo = pl.ds(k * 2 * LANES, LANES)
                sl_hi = pl.ds(k * 2 * LANES + LANES, LANES)
                plsc.addupdate(srcbuf_v.at[b, r, sl_lo], lo)
                plsc.addupdate(srcbuf_v.at[b, r, sl_hi], hi)

    start_src(0, 0)

    pltpu.sync_copy(sil_hbm.at[pl.ds(base, B_PER_W)], sil_v)

    for k in range(B_PER_W // LANES):
        s = sil_v[pl.ds(k * LANES, LANES)]
        idx_v[pl.ds(k * LANES, LANES)] = jnp.where(
            s > 0, jnp.minimum(s, MAX_LEN - 1), MAX_LEN
        )

    start_pe(0, 0)
    for c in range(1, NBUF):
        start_src(c, c)
    for c in range(1, PEBUF):
        start_pe(c, c)

    def wait_store(c, b):
        pltpu.make_async_copy(srcbuf_v.at[b],
                              out_hbm.at[pl.ds(base + c * CHUNK, CHUNK)],
                              sem_out.at[b]).wait()

    def chunk_body(c, _):
        cur = jnp.bitwise_and(c, NBUF - 1)
        prv = jnp.bitwise_and(c - 1, NBUF - 1)

        @pl.when(jnp.logical_and(c >= 1, c - 1 + NBUF < N_CHUNKS))
        def _():
            wait_store(c - 1, prv)
            start_src(c - 1 + NBUF, prv)

        @pl.when(jnp.logical_and(c >= 1, c - 1 + PEBUF < N_CHUNKS))
        def _():
            start_pe(c - 1 + PEBUF, lax.rem(c - 1 + PEBUF, PEBUF))

        wait_loads(c, cur, lax.rem(c, PEBUF))
        do_add(c, cur)
        pltpu.async_copy(srcbuf_v.at[cur],
                         out_hbm.at[pl.ds(base + c * CHUNK, CHUNK)],
                         sem_out.at[cur])
        return 0

    lax.fori_loop(0, N_CHUNKS, chunk_body, 0)
    for c in range(max(0, N_CHUNKS - NBUF), N_CHUNKS):
        wait_store(c, c % NBUF)


@jax.jit
def _run(src2d, sil, pe_bf16):
    mesh = plsc.VectorSubcoreMesh(core_axis_name="c", subcore_axis_name="s")
    fn = pl.kernel(
        _sc_body,
        out_type=jax.ShapeDtypeStruct((SEQ, D_MODEL), jnp.float32),
        mesh=mesh,
        scratch_types=[
            pltpu.VMEM((B_PER_W,), jnp.int32),
            pltpu.VMEM((B_PER_W,), jnp.int32),
            pltpu.VMEM((NBUF, CHUNK, D_MODEL), jnp.float32),
            pltpu.VMEM((PEBUF, CHUNK, D_MODEL // 2), jnp.int32),
            pltpu.SemaphoreType.DMA((NBUF,)),
            pltpu.SemaphoreType.DMA((PEBUF,)),
            pltpu.SemaphoreType.DMA((NBUF,)),
        ],
    )
    return fn(src2d, sil, pe_bf16)


def kernel(src, silence, pe):
    src2d = src.reshape(SEQ, D_MODEL)
    sil = silence.astype(jnp.int32)
    pe_bf16 = (
        pe.astype(jnp.float32)
        .reshape(MAX_LEN, D_MODEL // 32, 2, LANES)
        .swapaxes(2, 3)
        .reshape(MAX_LEN, D_MODEL // 2, 2)
        .astype(jnp.bfloat16)
    )
    pe_i32 = jnp.concatenate(
        [
            jax.lax.bitcast_convert_type(pe_bf16, jnp.int32),
            jnp.zeros((1, D_MODEL // 2), jnp.int32),
        ],
        axis=0,
    )
    out = _run(src2d, sil, pe_i32)
    return out.reshape(1, SEQ, D_MODEL)
